# baseline (speedup 1.0000x reference)
import functools

import jax
import jax.numpy as jnp
from jax import lax
from jax.experimental import pallas as pl
from jax.experimental.pallas import tpu as pltpu
from jax.experimental.pallas import tpu_sc as plsc

_BATCH = 16384
_FDIM = 2048
_MASK = 256

_NC = 2
_NS = 16
_NW = _NC * _NS
_ROWS_PER_W = _BATCH // _NW
_CH = 16
_CHE = _CH * _FDIM
_NBUF = 2
_LOOK = 1
_NCHUNK = _ROWS_PER_W // _CH
_NLAP = _NCHUNK // _NBUF
_SIDX = _CH * _MASK


def _sc_kernel(x_hbm, idx_hbm, out_hbm, idx_v, sidx_v, b0, b1,
               si0, si1, so0, so1):
    bufs = (b0, b1)
    sin = (si0, si1)
    sout = (so0, so1)

    wid = lax.axis_index("c") * _NS + lax.axis_index("s")
    elem0 = wid * (_ROWS_PER_W * _FDIM)

    pltpu.sync_copy(idx_hbm, idx_v)
    zeros16 = jnp.zeros((16,), jnp.float32)

    for r in range(_CH):
        for k in range(_MASK // 16):
            sidx_v[pl.ds(r * _MASK + k * 16, 16)] = (
                idx_v[pl.ds(k * 16, 16)] + r * _FDIM
            )

    def start_in(c, b):
        pltpu.make_async_copy(
            x_hbm.at[pl.ds(elem0 + c * _CHE, _CHE)], bufs[b], sin[b]
        ).start()

    def wait_in(b):
        pltpu.make_async_copy(
            x_hbm.at[pl.ds(elem0, _CHE)], bufs[b], sin[b]
        ).wait()

    def start_out(c, b):
        pltpu.make_async_copy(
            bufs[b], out_hbm.at[pl.ds(elem0 + c * _CHE, _CHE)], sout[b]
        ).start()

    def wait_out(b):
        pltpu.make_async_copy(
            bufs[b], out_hbm.at[pl.ds(elem0, _CHE)], sout[b]
        ).wait()

    def scatter_zeros(b):
        buf = bufs[b]
        for t in range(_SIDX // 16):
            plsc.store_scatter(buf, [sidx_v[pl.ds(t * 16, 16)]], zeros16)

    def chunk_iter(c, j, do_wait_out, do_reload):
        if do_reload:
            bf = (j + _LOOK) % _NBUF
            if do_wait_out:
                wait_out(bf)
            start_in(c + _LOOK, bf)
        wait_in(j)
        scatter_zeros(j)
        start_out(c, j)

    for c in range(_LOOK):
        start_in(c, c)

    for j in range(_NBUF):
        chunk_iter(j, j, do_wait_out=(j >= _NBUF - _LOOK), do_reload=True)

    def lap(i, carry):
        for j in range(_NBUF):
            chunk_iter(i * _NBUF + j, j, do_wait_out=True, do_reload=True)
        return carry

    lax.fori_loop(1, _NLAP - 1, lap, 0)

    for j in range(_NBUF):
        c = (_NLAP - 1) * _NBUF + j
        chunk_iter(c, j, do_wait_out=(j < _NBUF - _LOOK),
                   do_reload=(j < _NBUF - _LOOK))

    for b in range(_NBUF):
        wait_out(b)


def kernel(x, mask_indices):
    mesh = plsc.VectorSubcoreMesh(core_axis_name="c", subcore_axis_name="s")
    f = functools.partial(
        pl.kernel,
        mesh=mesh,
        out_type=jax.ShapeDtypeStruct((_BATCH * _FDIM,), jnp.float32),
        scratch_types=[
            pltpu.VMEM((_MASK,), jnp.int32),
            pltpu.VMEM((_SIDX,), jnp.int32),
        ] + [pltpu.VMEM((_CHE,), jnp.float32) for _ in range(_NBUF)]
        + [pltpu.SemaphoreType.DMA for _ in range(2 * _NBUF)],
        compiler_params=pltpu.CompilerParams(needs_layout_passes=False),
    )(_sc_kernel)
    out = f(x.reshape(-1), mask_indices)
    return out.reshape(_BATCH, _FDIM)


# speedup vs baseline: 3.4202x; 3.4202x over previous
import functools

import jax
import jax.numpy as jnp
from jax import lax
from jax.experimental import pallas as pl
from jax.experimental.pallas import tpu as pltpu
from jax.experimental.pallas import tpu_sc as plsc

_BATCH = 16384
_FDIM = 2048
_MASK = 256
_BR = 1024

_NS = 16


def _mask_sc_kernel(idx_hbm, mask_hbm, idx_v, mask_v):
    wid = lax.axis_index("c") * _NS + lax.axis_index("s")

    @pl.when(wid == 0)
    def _():
        pltpu.sync_copy(idx_hbm, idx_v)
        ones16 = jnp.ones((16,), jnp.float32)
        zeros16 = jnp.zeros((16,), jnp.float32)
        for i in range(_FDIM // 16):
            mask_v[pl.ds(i * 16, 16)] = ones16
        for k in range(_MASK // 16):
            plsc.store_scatter(mask_v, [idx_v[pl.ds(k * 16, 16)]], zeros16)
        pltpu.sync_copy(mask_v, mask_hbm)


def _sc_mask(mask_indices):
    mesh = plsc.VectorSubcoreMesh(core_axis_name="c", subcore_axis_name="s")
    f = functools.partial(
        pl.kernel,
        mesh=mesh,
        out_type=jax.ShapeDtypeStruct((_FDIM,), jnp.float32),
        scratch_types=[
            pltpu.VMEM((_MASK,), jnp.int32),
            pltpu.VMEM((_FDIM,), jnp.float32),
        ],
        compiler_params=pltpu.CompilerParams(needs_layout_passes=False),
    )(_mask_sc_kernel)
    return f(mask_indices)


def _tc_body(x_ref, m_ref, o_ref):
    o_ref[...] = x_ref[...] * m_ref[...]


def _tc_call(x, mask_row):
    grid = (_BATCH // _BR,)
    return pl.pallas_call(
        _tc_body,
        grid=grid,
        in_specs=[
            pl.BlockSpec((_BR, _FDIM), lambda i: (i, 0)),
            pl.BlockSpec((1, _FDIM), lambda i: (0, 0)),
        ],
        out_specs=pl.BlockSpec((_BR, _FDIM), lambda i: (i, 0)),
        out_shape=jax.ShapeDtypeStruct((_BATCH, _FDIM), jnp.float32),
        compiler_params=pltpu.CompilerParams(
            dimension_semantics=("arbitrary",),
        ),
    )(x, mask_row)


def kernel(x, mask_indices):
    mask = _sc_mask(mask_indices)
    return _tc_call(x, mask.reshape(1, _FDIM))
